# Initial kernel scaffold; baseline (speedup 1.0000x reference)
import jax
import jax.numpy as jnp
from jax.experimental import pallas as pl


def kernel(x, edge_index, layer, W_att, b_att, node_anchor, W_ew, b_ew, edge_anchor, W_c1, b_c1, W_c2, b_c2, W_i, b_i):
    raise NotImplementedError("write your pallas kernel here")


# speedup vs baseline: 5.4398x; 5.4398x over previous
import functools

import jax
import jax.numpy as jnp
from jax import lax
from jax.experimental import pallas as pl
from jax.experimental.pallas import tpu as pltpu
from jax.experimental.pallas import tpu_sc as plsc

N, E, D, A = 10000, 320000, 128, 5
NP = 10240
CHUNK = 80
NEG = -1e30


def _tab_body(x_ref, wcat_ref, bcat_ref, tab_ref):
    tab_ref[...] = (
        jnp.dot(x_ref[...], wcat_ref[...], preferred_element_type=jnp.float32)
        + bcat_ref[...]
    )


def _edge_prompt_body(brows_ref, eap_ref, out_ref):
    out_ref[...] = jnp.dot(
        brows_ref[...], eap_ref[...], preferred_element_type=jnp.float32
    )


def _final_body(x_ref, cp_ref, wap_ref, bap_ref, nap_ref, eap_ref, w1a_ref,
                w1b_ref, bc1_ref, wc2_ref, bc2_ref, wi_ref, bi_ref, out_ref):
    f32 = jnp.float32
    xb = x_ref[...]
    scores = jnp.dot(xb, wap_ref[...], preferred_element_type=f32) + bap_ref[...]
    w = jax.nn.softmax(scores, axis=1)
    npx = xb + jnp.dot(w, nap_ref[...], preferred_element_type=f32)
    coef = jnp.sum(cp_ref[...], axis=0)
    agg = lax.dot_general(coef, eap_ref[...], (((0,), (0,)), ((), ())),
                          preferred_element_type=f32)
    h = jnp.maximum(
        jnp.dot(npx, w1a_ref[...], preferred_element_type=f32)
        + jnp.dot(agg, w1b_ref[...], preferred_element_type=f32)
        + bc1_ref[...], 0.0)
    cstr = jax.nn.sigmoid(
        jnp.dot(h, wc2_ref[...], preferred_element_type=f32) + bc2_ref[...])
    itv = (jnp.dot(npx + agg, wi_ref[...], preferred_element_type=f32)
           + bi_ref[...])
    out_ref[...] = npx + cstr * itv


def _edge_sc(tab, src1d, dst1d):
    info = plsc.get_sparse_core_info()
    NC, NS, L = info.num_cores, info.num_subcores, info.num_lanes
    NW = NC * NS
    ept = E // NW
    nch = ept // CHUNK
    G = CHUNK // L
    CT = 8 * NP

    mesh = plsc.VectorSubcoreMesh(core_axis_name="c", subcore_axis_name="s")

    @functools.partial(
        pl.kernel, mesh=mesh,
        compiler_params=pltpu.CompilerParams(
            needs_layout_passes=False, use_tc_tiling_on_sc=False),
        out_type=(
            jax.ShapeDtypeStruct((E * 16,), jnp.float32),
            jax.ShapeDtypeStruct((NW, CT), jnp.float32),
        ),
        scratch_types=[
            pltpu.VMEM((ept,), jnp.int32),
            pltpu.VMEM((ept,), jnp.int32),
            pltpu.VMEM((CHUNK, 16), jnp.float32),
            pltpu.VMEM((CHUNK, 16), jnp.float32),
            pltpu.VMEM((CHUNK * 16,), jnp.float32),
            pltpu.VMEM((CHUNK * 16,), jnp.float32),
            pltpu.VMEM((CHUNK * 16,), jnp.float32),
            pltpu.VMEM((CT,), jnp.float32),
            pltpu.SemaphoreType.DMA,
            pltpu.SemaphoreType.DMA,
        ],
    )
    def k(tab_h, src_h, dst_h, brows_h, coef_h,
          sidx, didx, srows, drows, st, dt, brow, coefT, sem1, sem2):
        wid = lax.axis_index("s") * NC + lax.axis_index("c")
        pltpu.sync_copy(src_h.at[pl.ds(wid * ept, ept)], sidx)
        pltpu.sync_copy(dst_h.at[pl.ds(wid * ept, ept)], didx)

        zero = jnp.zeros((L,), jnp.float32)

        def zb(i, carry):
            brow[pl.ds(i * L, L)] = zero
            return carry

        lax.fori_loop(0, (CHUNK * 16) // L, zb, 0)

        def zc(i, carry):
            coefT[pl.ds(i * L, L)] = zero
            return carry

        lax.fori_loop(0, CT // L, zc, 0)

        iota = lax.iota(jnp.int32, L)

        def chunk_body(c, carry):
            cp1 = pltpu.async_copy(
                tab_h.at[sidx.at[pl.ds(c * CHUNK, CHUNK)]], srows, sem1)
            cp2 = pltpu.async_copy(
                tab_h.at[didx.at[pl.ds(c * CHUNK, CHUNK)]], drows, sem2)
            cp1.wait()
            cp2.wait()

            def tr(e, carry):
                plsc.store_scatter(st, [iota * CHUNK + e], srows[e, :])
                plsc.store_scatter(dt, [iota * CHUNK + e], drows[e, :])
                return carry

            lax.fori_loop(0, CHUNK, tr, 0)
            for g in range(G):
                ridx = iota + g * L
                sa = [st[pl.ds(a * CHUNK + g * L, L)] for a in range(A)]
                sb = [dt[pl.ds((a + 5) * CHUNK + g * L, L)] for a in range(A)]
                lg = [v + u for v, u in zip(sa, sb)]
                lg = [jnp.maximum(v, v * 0.01) for v in lg]
                m = jnp.maximum(jnp.maximum(jnp.maximum(lg[0], lg[1]),
                                            jnp.maximum(lg[2], lg[3])), lg[4])
                ex = [jnp.exp(v - m) for v in lg]
                tot = ((ex[0] + ex[1]) + (ex[2] + ex[3])) + ex[4]
                inv = jnp.float32(1.0) / tot
                bb = [e * inv for e in ex]
                si = sidx[pl.ds(c * CHUNK + g * L, L)]
                di = didx[pl.ds(c * CHUNK + g * L, L)]
                for a in range(A):
                    plsc.store_scatter(brow, [ridx * 16 + a], bb[a])
                    plsc.addupdate_scatter(coefT, [si + a * NP], bb[a])
                    plsc.addupdate_scatter(coefT, [di + a * NP], bb[a])
            pltpu.sync_copy(
                brow,
                brows_h.at[pl.ds((wid * ept + c * CHUNK) * 16, CHUNK * 16)])
            return carry

        lax.fori_loop(0, nch, chunk_body, 0)
        pltpu.sync_copy(coefT, coef_h.at[wid])

    return k(tab, src1d, dst1d)


def kernel(x, edge_index, layer, W_att, b_att, node_anchor, W_ew, b_ew,
           edge_anchor, W_c1, b_c1, W_c2, b_c2, W_i, b_i):
    f32 = jnp.float32
    wcat = jnp.concatenate(
        [W_ew[:D], W_ew[D:], jnp.zeros((D, 6), f32)], axis=1)
    bcat = jnp.concatenate([b_ew, jnp.zeros((11,), f32)])[None, :]
    wap = jnp.concatenate([W_att, jnp.zeros((D, 3), f32)], axis=1)
    bap = jnp.concatenate([b_att, jnp.full((3,), NEG, f32)])[None, :]
    nap = jnp.concatenate([node_anchor, jnp.zeros((3, D), f32)], axis=0)
    eap8 = jnp.concatenate([edge_anchor, jnp.zeros((3, D), f32)], axis=0)
    eap16 = jnp.concatenate([edge_anchor, jnp.zeros((11, D), f32)], axis=0)
    w1a, w1b = W_c1[:D], W_c1[D:]
    bc1 = b_c1[None, :]
    bc2 = b_c2[None, :]
    bi = b_i[None, :]
    src1d = edge_index[0]
    dst1d = edge_index[1]

    tab = pl.pallas_call(
        _tab_body,
        grid=(5,),
        in_specs=[
            pl.BlockSpec((2000, D), lambda i: (i, 0)),
            pl.BlockSpec((D, 16), lambda i: (0, 0)),
            pl.BlockSpec((1, 16), lambda i: (0, 0)),
        ],
        out_specs=pl.BlockSpec((2000, 16), lambda i: (i, 0)),
        out_shape=jax.ShapeDtypeStruct((N, 16), f32),
    )(x, wcat, bcat)

    brows_flat, coef_flat = _edge_sc(tab, src1d, dst1d)
    brows = brows_flat.reshape(E, 16)
    coef_part = coef_flat.reshape(-1, 8, NP)

    edge_prompt = pl.pallas_call(
        _edge_prompt_body,
        grid=(160,),
        in_specs=[
            pl.BlockSpec((2000, 16), lambda i: (i, 0)),
            pl.BlockSpec((16, D), lambda i: (0, 0)),
        ],
        out_specs=pl.BlockSpec((2000, D), lambda i: (i, 0)),
        out_shape=jax.ShapeDtypeStruct((E, D), f32),
    )(brows, eap16)

    NWP = coef_part.shape[0]
    nblk = (N + D - 1) // D
    final_x = pl.pallas_call(
        _final_body,
        grid=(nblk,),
        in_specs=[
            pl.BlockSpec((D, D), lambda i: (i, 0)),
            pl.BlockSpec((NWP, 8, D), lambda i: (0, 0, i)),
            pl.BlockSpec((D, 8), lambda i: (0, 0)),
            pl.BlockSpec((1, 8), lambda i: (0, 0)),
            pl.BlockSpec((8, D), lambda i: (0, 0)),
            pl.BlockSpec((8, D), lambda i: (0, 0)),
            pl.BlockSpec((D, D), lambda i: (0, 0)),
            pl.BlockSpec((D, D), lambda i: (0, 0)),
            pl.BlockSpec((1, D), lambda i: (0, 0)),
            pl.BlockSpec((D, D), lambda i: (0, 0)),
            pl.BlockSpec((1, D), lambda i: (0, 0)),
            pl.BlockSpec((D, D), lambda i: (0, 0)),
            pl.BlockSpec((1, D), lambda i: (0, 0)),
        ],
        out_specs=pl.BlockSpec((D, D), lambda i: (i, 0)),
        out_shape=jax.ShapeDtypeStruct((N, D), f32),
    )(x, coef_part, wap, bap, nap, eap8, w1a, w1b, bc1, W_c2, bc2, W_i, bi)

    return (final_x, edge_prompt)


# speedup vs baseline: 6.4441x; 1.1846x over previous
import functools

import jax
import jax.numpy as jnp
from jax import lax
from jax.experimental import pallas as pl
from jax.experimental.pallas import tpu as pltpu
from jax.experimental.pallas import tpu_sc as plsc

N, E, D, A = 10000, 320000, 128, 5
NP = 10240
CHUNK = 80
NEG = -1e30


def _tab_body(x_ref, wcat_ref, bcat_ref, tab_ref):
    tab_ref[...] = (
        jnp.dot(x_ref[...], wcat_ref[...], preferred_element_type=jnp.float32)
        + bcat_ref[...]
    )


def _edge_prompt_body(brows_ref, eap_ref, out_ref):
    out_ref[...] = jnp.dot(
        brows_ref[...], eap_ref[...], preferred_element_type=jnp.float32
    )


def _final_body(x_ref, cp_ref, wap_ref, bap_ref, nap_ref, eap_ref, w1a_ref,
                w1b_ref, bc1_ref, wc2_ref, bc2_ref, wi_ref, bi_ref, out_ref):
    f32 = jnp.float32
    xb = x_ref[...]
    scores = jnp.dot(xb, wap_ref[...], preferred_element_type=f32) + bap_ref[...]
    w = jax.nn.softmax(scores, axis=1)
    npx = xb + jnp.dot(w, nap_ref[...], preferred_element_type=f32)
    coef = jnp.sum(cp_ref[...], axis=0)
    agg = lax.dot_general(coef, eap_ref[...], (((0,), (0,)), ((), ())),
                          preferred_element_type=f32)
    h = jnp.maximum(
        jnp.dot(npx, w1a_ref[...], preferred_element_type=f32)
        + jnp.dot(agg, w1b_ref[...], preferred_element_type=f32)
        + bc1_ref[...], 0.0)
    cstr = jax.nn.sigmoid(
        jnp.dot(h, wc2_ref[...], preferred_element_type=f32) + bc2_ref[...])
    itv = (jnp.dot(npx + agg, wi_ref[...], preferred_element_type=f32)
           + bi_ref[...])
    out_ref[...] = npx + cstr * itv


def _edge_sc(tab, src1d, dst1d):
    info = plsc.get_sparse_core_info()
    NC, NS, L = info.num_cores, info.num_subcores, info.num_lanes
    NW = NC * NS
    ept = E // NW
    nch = ept // CHUNK
    G = CHUNK // L
    CT = 8 * NP

    mesh = plsc.VectorSubcoreMesh(core_axis_name="c", subcore_axis_name="s")

    @functools.partial(
        pl.kernel, mesh=mesh,
        compiler_params=pltpu.CompilerParams(
            needs_layout_passes=False, use_tc_tiling_on_sc=False),
        out_type=(
            jax.ShapeDtypeStruct((E * 16,), jnp.float32),
            jax.ShapeDtypeStruct((NW, CT), jnp.float32),
        ),
        scratch_types=[
            pltpu.VMEM((ept,), jnp.int32),
            pltpu.VMEM((ept,), jnp.int32),
            pltpu.VMEM((CHUNK, 16), jnp.float32),
            pltpu.VMEM((CHUNK, 16), jnp.float32),
            pltpu.VMEM((CHUNK, 16), jnp.float32),
            pltpu.VMEM((CHUNK, 16), jnp.float32),
            pltpu.VMEM((CHUNK * 16,), jnp.float32),
            pltpu.VMEM((CHUNK * 16,), jnp.float32),
            pltpu.VMEM((CHUNK * 16,), jnp.float32),
            pltpu.VMEM((CHUNK * 16,), jnp.float32),
            pltpu.VMEM((CT,), jnp.float32),
            pltpu.SemaphoreType.DMA,
            pltpu.SemaphoreType.DMA,
            pltpu.SemaphoreType.DMA,
            pltpu.SemaphoreType.DMA,
            pltpu.SemaphoreType.DMA,
            pltpu.SemaphoreType.DMA,
        ],
    )
    def k(tab_h, src_h, dst_h, brows_h, coef_h,
          sidx, didx, srows0, srows1, drows0, drows1, st, dt, brow0, brow1,
          coefT, sa0, sa1, sb0, sb1, sw0, sw1):
        wid = lax.axis_index("s") * NC + lax.axis_index("c")
        pltpu.sync_copy(src_h.at[pl.ds(wid * ept, ept)], sidx)
        pltpu.sync_copy(dst_h.at[pl.ds(wid * ept, ept)], didx)

        zero = jnp.zeros((L,), jnp.float32)

        def zb(i, carry):
            brow0[pl.ds(i * L, L)] = zero
            brow1[pl.ds(i * L, L)] = zero
            return carry

        lax.fori_loop(0, (CHUNK * 16) // L, zb, 0)

        def zc(i, carry):
            coefT[pl.ds(i * L, L)] = zero
            return carry

        lax.fori_loop(0, CT // L, zc, 0)

        iota = lax.iota(jnp.int32, L)

        def issue(c, sbuf, dbuf, ssem, dsem):
            pltpu.async_copy(
                tab_h.at[sidx.at[pl.ds(c * CHUNK, CHUNK)]], sbuf, ssem)
            pltpu.async_copy(
                tab_h.at[didx.at[pl.ds(c * CHUNK, CHUNK)]], dbuf, dsem)

        def wait_rows(c, sbuf, dbuf, ssem, dsem):
            pltpu.make_async_copy(
                tab_h.at[sidx.at[pl.ds(c * CHUNK, CHUNK)]], sbuf, ssem).wait()
            pltpu.make_async_copy(
                tab_h.at[didx.at[pl.ds(c * CHUNK, CHUNK)]], dbuf, dsem).wait()

        def brow_dst(c):
            return brows_h.at[pl.ds((wid * ept + c * CHUNK) * 16, CHUNK * 16)]

        def compute(c, sbuf, dbuf, browbuf, wsem):
            def tr(e, carry):
                plsc.store_scatter(st, [iota * CHUNK + e], sbuf[e, :])
                plsc.store_scatter(dt, [iota * CHUNK + e], dbuf[e, :])
                return carry

            lax.fori_loop(0, CHUNK, tr, 0)
            for g in range(G):
                ridx = iota + g * L
                sa = [st[pl.ds(a * CHUNK + g * L, L)] for a in range(A)]
                sb = [dt[pl.ds((a + 5) * CHUNK + g * L, L)] for a in range(A)]
                lg = [v + u for v, u in zip(sa, sb)]
                lg = [jnp.maximum(v, v * 0.01) for v in lg]
                m = jnp.maximum(jnp.maximum(jnp.maximum(lg[0], lg[1]),
                                            jnp.maximum(lg[2], lg[3])), lg[4])
                ex = [jnp.exp(v - m) for v in lg]
                tot = ((ex[0] + ex[1]) + (ex[2] + ex[3])) + ex[4]
                inv = jnp.float32(1.0) / tot
                bb = [e * inv for e in ex]
                si = sidx[pl.ds(c * CHUNK + g * L, L)]
                di = didx[pl.ds(c * CHUNK + g * L, L)]
                for a in range(A):
                    plsc.store_scatter(browbuf, [ridx * 16 + a], bb[a])
                    plsc.addupdate_scatter(coefT, [si + a * NP], bb[a])
                    plsc.addupdate_scatter(coefT, [di + a * NP], bb[a])
            pltpu.async_copy(browbuf, brow_dst(c), wsem)

        def wait_brow(c_prev, browbuf, wsem):
            pltpu.make_async_copy(browbuf, brow_dst(c_prev), wsem).wait()

        issue(0, srows0, drows0, sa0, sb0)

        def pair_body(kk, carry):
            c0 = 2 * kk
            issue(c0 + 1, srows1, drows1, sa1, sb1)
            wait_rows(c0, srows0, drows0, sa0, sb0)

            @pl.when(kk > 0)
            def _():
                wait_brow(c0 - 2, brow0, sw0)

            compute(c0, srows0, drows0, brow0, sw0)
            issue(c0 + 2, srows0, drows0, sa0, sb0)
            wait_rows(c0 + 1, srows1, drows1, sa1, sb1)

            @pl.when(kk > 0)
            def _():
                wait_brow(c0 - 1, brow1, sw1)

            compute(c0 + 1, srows1, drows1, brow1, sw1)
            return carry

        lax.fori_loop(0, (nch - 1) // 2, pair_body, 0)
        last = nch - 1
        wait_rows(last, srows0, drows0, sa0, sb0)
        wait_brow(last - 2, brow0, sw0)
        compute(last, srows0, drows0, brow0, sw0)
        wait_brow(last - 1, brow1, sw1)
        wait_brow(last, brow0, sw0)
        pltpu.sync_copy(coefT, coef_h.at[wid])

    return k(tab, src1d, dst1d)


def kernel(x, edge_index, layer, W_att, b_att, node_anchor, W_ew, b_ew,
           edge_anchor, W_c1, b_c1, W_c2, b_c2, W_i, b_i):
    f32 = jnp.float32
    wcat = jnp.concatenate(
        [W_ew[:D], W_ew[D:], jnp.zeros((D, 6), f32)], axis=1)
    bcat = jnp.concatenate([b_ew, jnp.zeros((11,), f32)])[None, :]
    wap = jnp.concatenate([W_att, jnp.zeros((D, 3), f32)], axis=1)
    bap = jnp.concatenate([b_att, jnp.full((3,), NEG, f32)])[None, :]
    nap = jnp.concatenate([node_anchor, jnp.zeros((3, D), f32)], axis=0)
    eap8 = jnp.concatenate([edge_anchor, jnp.zeros((3, D), f32)], axis=0)
    eap16 = jnp.concatenate([edge_anchor, jnp.zeros((11, D), f32)], axis=0)
    w1a, w1b = W_c1[:D], W_c1[D:]
    bc1 = b_c1[None, :]
    bc2 = b_c2[None, :]
    bi = b_i[None, :]
    src1d = edge_index[0]
    dst1d = edge_index[1]

    tab = pl.pallas_call(
        _tab_body,
        grid=(5,),
        in_specs=[
            pl.BlockSpec((2000, D), lambda i: (i, 0)),
            pl.BlockSpec((D, 16), lambda i: (0, 0)),
            pl.BlockSpec((1, 16), lambda i: (0, 0)),
        ],
        out_specs=pl.BlockSpec((2000, 16), lambda i: (i, 0)),
        out_shape=jax.ShapeDtypeStruct((N, 16), f32),
    )(x, wcat, bcat)

    brows_flat, coef_flat = _edge_sc(tab, src1d, dst1d)
    brows = brows_flat.reshape(E, 16)
    coef_part = coef_flat.reshape(-1, 8, NP)

    edge_prompt = pl.pallas_call(
        _edge_prompt_body,
        grid=(160,),
        in_specs=[
            pl.BlockSpec((2000, 16), lambda i: (i, 0)),
            pl.BlockSpec((16, D), lambda i: (0, 0)),
        ],
        out_specs=pl.BlockSpec((2000, D), lambda i: (i, 0)),
        out_shape=jax.ShapeDtypeStruct((E, D), f32),
    )(brows, eap16)

    NWP = coef_part.shape[0]
    nblk = (N + D - 1) // D
    final_x = pl.pallas_call(
        _final_body,
        grid=(nblk,),
        in_specs=[
            pl.BlockSpec((D, D), lambda i: (i, 0)),
            pl.BlockSpec((NWP, 8, D), lambda i: (0, 0, i)),
            pl.BlockSpec((D, 8), lambda i: (0, 0)),
            pl.BlockSpec((1, 8), lambda i: (0, 0)),
            pl.BlockSpec((8, D), lambda i: (0, 0)),
            pl.BlockSpec((8, D), lambda i: (0, 0)),
            pl.BlockSpec((D, D), lambda i: (0, 0)),
            pl.BlockSpec((D, D), lambda i: (0, 0)),
            pl.BlockSpec((1, D), lambda i: (0, 0)),
            pl.BlockSpec((D, D), lambda i: (0, 0)),
            pl.BlockSpec((1, D), lambda i: (0, 0)),
            pl.BlockSpec((D, D), lambda i: (0, 0)),
            pl.BlockSpec((1, D), lambda i: (0, 0)),
        ],
        out_specs=pl.BlockSpec((D, D), lambda i: (i, 0)),
        out_shape=jax.ShapeDtypeStruct((N, D), f32),
    )(x, coef_part, wap, bap, nap, eap8, w1a, w1b, bc1, W_c2, bc2, W_i, bi)

    return (final_x, edge_prompt)


# speedup vs baseline: 9.0926x; 1.4110x over previous
import functools

import jax
import jax.numpy as jnp
from jax import lax
from jax.experimental import pallas as pl
from jax.experimental.pallas import tpu as pltpu
from jax.experimental.pallas import tpu_sc as plsc

N, E, D, A = 10000, 320000, 128, 5
NP = 10240
CHUNK = 80
NEG = -1e30


def _tab_body(x_ref, wcat_ref, bcat_ref, tab_ref):
    tab_ref[...] = (
        jnp.dot(x_ref[...], wcat_ref[...], preferred_element_type=jnp.float32)
        + bcat_ref[...]
    )


def _edge_prompt_body(brows_ref, wbd_ref, out_ref):
    t = jnp.dot(brows_ref[...], wbd_ref[...],
                preferred_element_type=jnp.float32)
    out_ref[...] = t.reshape(out_ref.shape)


def _final_body(x_ref, cp_ref, wap_ref, bap_ref, nap_ref, eap_ref, w1a_ref,
                w1b_ref, bc1_ref, wc2_ref, bc2_ref, wi_ref, bi_ref, out_ref):
    f32 = jnp.float32
    xb = x_ref[...]
    scores = jnp.dot(xb, wap_ref[...], preferred_element_type=f32) + bap_ref[...]
    w = jax.nn.softmax(scores, axis=1)
    npx = xb + jnp.dot(w, nap_ref[...], preferred_element_type=f32)
    coef = jnp.sum(cp_ref[...], axis=0)
    agg = lax.dot_general(coef, eap_ref[...], (((0,), (0,)), ((), ())),
                          preferred_element_type=f32)
    h = jnp.maximum(
        jnp.dot(npx, w1a_ref[...], preferred_element_type=f32)
        + jnp.dot(agg, w1b_ref[...], preferred_element_type=f32)
        + bc1_ref[...], 0.0)
    cstr = jax.nn.sigmoid(
        jnp.dot(h, wc2_ref[...], preferred_element_type=f32) + bc2_ref[...])
    itv = (jnp.dot(npx + agg, wi_ref[...], preferred_element_type=f32)
           + bi_ref[...])
    out_ref[...] = npx + cstr * itv


def _edge_sc(tab, src1d, dst1d):
    info = plsc.get_sparse_core_info()
    NC, NS, L = info.num_cores, info.num_subcores, info.num_lanes
    NW = NC * NS
    ept = E // NW
    nch = ept // CHUNK
    G = CHUNK // L
    CT = 8 * NP

    mesh = plsc.VectorSubcoreMesh(core_axis_name="c", subcore_axis_name="s")

    @functools.partial(
        pl.kernel, mesh=mesh,
        compiler_params=pltpu.CompilerParams(
            needs_layout_passes=False, use_tc_tiling_on_sc=False),
        out_type=(
            jax.ShapeDtypeStruct((E * 16,), jnp.float32),
            jax.ShapeDtypeStruct((NW, CT), jnp.float32),
        ),
        scratch_types=[
            pltpu.VMEM((ept,), jnp.int32),
            pltpu.VMEM((ept,), jnp.int32),
            pltpu.VMEM((CHUNK, 16), jnp.float32),
            pltpu.VMEM((CHUNK, 16), jnp.float32),
            pltpu.VMEM((CHUNK, 16), jnp.float32),
            pltpu.VMEM((CHUNK, 16), jnp.float32),
            pltpu.VMEM((CHUNK * 16,), jnp.float32),
            pltpu.VMEM((CHUNK * 16,), jnp.float32),
            pltpu.VMEM((CHUNK * 16,), jnp.float32),
            pltpu.VMEM((CHUNK * 16,), jnp.float32),
            pltpu.VMEM((CT,), jnp.float32),
            pltpu.SemaphoreType.DMA,
            pltpu.SemaphoreType.DMA,
            pltpu.SemaphoreType.DMA,
            pltpu.SemaphoreType.DMA,
            pltpu.SemaphoreType.DMA,
            pltpu.SemaphoreType.DMA,
        ],
    )
    def k(tab_h, src_h, dst_h, brows_h, coef_h,
          sidx, didx, srows0, srows1, drows0, drows1, st, dt, brow0, brow1,
          coefT, sa0, sa1, sb0, sb1, sw0, sw1):
        wid = lax.axis_index("s") * NC + lax.axis_index("c")
        pltpu.sync_copy(src_h.at[pl.ds(wid * ept, ept)], sidx)
        pltpu.sync_copy(dst_h.at[pl.ds(wid * ept, ept)], didx)

        zero = jnp.zeros((L,), jnp.float32)

        def zb(i, carry):
            brow0[pl.ds(i * L, L)] = zero
            brow1[pl.ds(i * L, L)] = zero
            return carry

        lax.fori_loop(0, (CHUNK * 16) // L, zb, 0)

        def zc(i, carry):
            coefT[pl.ds(i * L, L)] = zero
            return carry

        lax.fori_loop(0, CT // L, zc, 0)

        iota = lax.iota(jnp.int32, L)

        def issue(c, sbuf, dbuf, ssem, dsem):
            pltpu.async_copy(
                tab_h.at[sidx.at[pl.ds(c * CHUNK, CHUNK)]], sbuf, ssem)
            pltpu.async_copy(
                tab_h.at[didx.at[pl.ds(c * CHUNK, CHUNK)]], dbuf, dsem)

        def wait_rows(c, sbuf, dbuf, ssem, dsem):
            pltpu.make_async_copy(
                tab_h.at[sidx.at[pl.ds(c * CHUNK, CHUNK)]], sbuf, ssem).wait()
            pltpu.make_async_copy(
                tab_h.at[didx.at[pl.ds(c * CHUNK, CHUNK)]], dbuf, dsem).wait()

        def brow_dst(c):
            return brows_h.at[pl.ds((wid * ept + c * CHUNK) * 16, CHUNK * 16)]

        def compute(c, sbuf, dbuf, browbuf, wsem):
            def tr(e, carry):
                plsc.store_scatter(st, [iota * CHUNK + e], sbuf[e, :])
                plsc.store_scatter(dt, [iota * CHUNK + e], dbuf[e, :])
                return carry

            lax.fori_loop(0, CHUNK, tr, 0)
            for g in range(G):
                ridx = iota + g * L
                sa = [st[pl.ds(a * CHUNK + g * L, L)] for a in range(A)]
                sb = [dt[pl.ds((a + 5) * CHUNK + g * L, L)] for a in range(A)]
                lg = [v + u for v, u in zip(sa, sb)]
                lg = [jnp.maximum(v, v * 0.01) for v in lg]
                m = jnp.maximum(jnp.maximum(jnp.maximum(lg[0], lg[1]),
                                            jnp.maximum(lg[2], lg[3])), lg[4])
                ex = [jnp.exp(v - m) for v in lg]
                tot = ((ex[0] + ex[1]) + (ex[2] + ex[3])) + ex[4]
                inv = jnp.float32(1.0) / tot
                bb = [e * inv for e in ex]
                si = sidx[pl.ds(c * CHUNK + g * L, L)]
                di = didx[pl.ds(c * CHUNK + g * L, L)]
                for a in range(A):
                    plsc.store_scatter(browbuf, [ridx * 16 + a], bb[a])
                    plsc.addupdate_scatter(coefT, [si + a * NP], bb[a])
                    plsc.addupdate_scatter(coefT, [di + a * NP], bb[a])
            pltpu.async_copy(browbuf, brow_dst(c), wsem)

        def wait_brow(c_prev, browbuf, wsem):
            pltpu.make_async_copy(browbuf, brow_dst(c_prev), wsem).wait()

        issue(0, srows0, drows0, sa0, sb0)

        def pair_body(kk, carry):
            c0 = 2 * kk
            issue(c0 + 1, srows1, drows1, sa1, sb1)
            wait_rows(c0, srows0, drows0, sa0, sb0)

            @pl.when(kk > 0)
            def _():
                wait_brow(c0 - 2, brow0, sw0)

            compute(c0, srows0, drows0, brow0, sw0)
            issue(c0 + 2, srows0, drows0, sa0, sb0)
            wait_rows(c0 + 1, srows1, drows1, sa1, sb1)

            @pl.when(kk > 0)
            def _():
                wait_brow(c0 - 1, brow1, sw1)

            compute(c0 + 1, srows1, drows1, brow1, sw1)
            return carry

        lax.fori_loop(0, (nch - 1) // 2, pair_body, 0)
        last = nch - 1
        wait_rows(last, srows0, drows0, sa0, sb0)
        wait_brow(last - 2, brow0, sw0)
        compute(last, srows0, drows0, brow0, sw0)
        wait_brow(last - 1, brow1, sw1)
        wait_brow(last, brow0, sw0)
        pltpu.sync_copy(coefT, coef_h.at[wid])

    return k(tab, src1d, dst1d)


def kernel(x, edge_index, layer, W_att, b_att, node_anchor, W_ew, b_ew,
           edge_anchor, W_c1, b_c1, W_c2, b_c2, W_i, b_i):
    f32 = jnp.float32
    wcat = jnp.concatenate(
        [W_ew[:D], W_ew[D:], jnp.zeros((D, 6), f32)], axis=1)
    bcat = jnp.concatenate([b_ew, jnp.zeros((11,), f32)])[None, :]
    wap = jnp.concatenate([W_att, jnp.zeros((D, 3), f32)], axis=1)
    bap = jnp.concatenate([b_att, jnp.full((3,), NEG, f32)])[None, :]
    nap = jnp.concatenate([node_anchor, jnp.zeros((3, D), f32)], axis=0)
    eap8 = jnp.concatenate([edge_anchor, jnp.zeros((3, D), f32)], axis=0)
    eap16 = jnp.concatenate([edge_anchor, jnp.zeros((11, D), f32)], axis=0)
    w1a, w1b = W_c1[:D], W_c1[D:]
    bc1 = b_c1[None, :]
    bc2 = b_c2[None, :]
    bi = b_i[None, :]
    src1d = edge_index[0]
    dst1d = edge_index[1]

    tab = pl.pallas_call(
        _tab_body,
        grid=(5,),
        in_specs=[
            pl.BlockSpec((2000, D), lambda i: (i, 0)),
            pl.BlockSpec((D, 16), lambda i: (0, 0)),
            pl.BlockSpec((1, 16), lambda i: (0, 0)),
        ],
        out_specs=pl.BlockSpec((2000, 16), lambda i: (i, 0)),
        out_shape=jax.ShapeDtypeStruct((N, 16), f32),
    )(x, wcat, bcat)

    brows_flat, coef_flat = _edge_sc(tab, src1d, dst1d)
    brows8 = brows_flat.reshape(E // 8, 8 * 16)
    coef_part = coef_flat.reshape(-1, 8, NP)
    wbd = jnp.kron(jnp.eye(8, dtype=f32), eap16)

    edge_prompt = pl.pallas_call(
        _edge_prompt_body,
        grid=(100,),
        in_specs=[
            pl.BlockSpec((400, D), lambda i: (i, 0)),
            pl.BlockSpec((D, 8 * D), lambda i: (0, 0)),
        ],
        out_specs=pl.BlockSpec((3200, D), lambda i: (i, 0)),
        out_shape=jax.ShapeDtypeStruct((E, D), f32),
    )(brows8, wbd)

    NWP = coef_part.shape[0]
    nblk = (N + D - 1) // D
    final_x = pl.pallas_call(
        _final_body,
        grid=(nblk,),
        in_specs=[
            pl.BlockSpec((D, D), lambda i: (i, 0)),
            pl.BlockSpec((NWP, 8, D), lambda i: (0, 0, i)),
            pl.BlockSpec((D, 8), lambda i: (0, 0)),
            pl.BlockSpec((1, 8), lambda i: (0, 0)),
            pl.BlockSpec((8, D), lambda i: (0, 0)),
            pl.BlockSpec((8, D), lambda i: (0, 0)),
            pl.BlockSpec((D, D), lambda i: (0, 0)),
            pl.BlockSpec((D, D), lambda i: (0, 0)),
            pl.BlockSpec((1, D), lambda i: (0, 0)),
            pl.BlockSpec((D, D), lambda i: (0, 0)),
            pl.BlockSpec((1, D), lambda i: (0, 0)),
            pl.BlockSpec((D, D), lambda i: (0, 0)),
            pl.BlockSpec((1, D), lambda i: (0, 0)),
        ],
        out_specs=pl.BlockSpec((D, D), lambda i: (i, 0)),
        out_shape=jax.ShapeDtypeStruct((N, D), f32),
    )(x, coef_part, wap, bap, nap, eap8, w1a, w1b, bc1, W_c2, bc2, W_i, bi)

    return (final_x, edge_prompt)


# speedup vs baseline: 9.5808x; 1.0537x over previous
import functools

import jax
import jax.numpy as jnp
from jax import lax
from jax.experimental import pallas as pl
from jax.experimental.pallas import tpu as pltpu
from jax.experimental.pallas import tpu_sc as plsc

N, E, D, A = 10000, 320000, 128, 5
NP = 10240
CHUNK = 80
NEG = -1e30


def _tab_body(x_ref, wcat_ref, bcat_ref, tab_ref):
    tab_ref[...] = (
        jnp.dot(x_ref[...], wcat_ref[...], preferred_element_type=jnp.float32)
        + bcat_ref[...]
    )


def _edge_prompt_body(brows_ref, wbd_ref, out_ref):
    t = jnp.dot(brows_ref[...], wbd_ref[...],
                preferred_element_type=jnp.float32)
    out_ref[...] = t.reshape(out_ref.shape)


def _final_body(x_ref, cp_ref, wap_ref, bap_ref, nap_ref, eap_ref, w1a_ref,
                w1b_ref, bc1_ref, wc2_ref, bc2_ref, wi_ref, bi_ref, out_ref):
    f32 = jnp.float32
    xb = x_ref[...]
    scores = jnp.dot(xb, wap_ref[...], preferred_element_type=f32) + bap_ref[...]
    w = jax.nn.softmax(scores, axis=1)
    npx = xb + jnp.dot(w, nap_ref[...], preferred_element_type=f32)
    coef = jnp.sum(cp_ref[...], axis=0)
    agg = lax.dot_general(coef, eap_ref[...], (((0,), (0,)), ((), ())),
                          preferred_element_type=f32)
    h = jnp.maximum(
        jnp.dot(npx, w1a_ref[...], preferred_element_type=f32)
        + jnp.dot(agg, w1b_ref[...], preferred_element_type=f32)
        + bc1_ref[...], 0.0)
    cstr = jax.nn.sigmoid(
        jnp.dot(h, wc2_ref[...], preferred_element_type=f32) + bc2_ref[...])
    itv = (jnp.dot(npx + agg, wi_ref[...], preferred_element_type=f32)
           + bi_ref[...])
    out_ref[...] = npx + cstr * itv


def _edge_sc(tab, src1d, dst1d):
    info = plsc.get_sparse_core_info()
    NC, NS, L = info.num_cores, info.num_subcores, info.num_lanes
    NW = NC * NS
    ept = E // NW
    nch = ept // CHUNK
    G = CHUNK // L
    CT = 8 * NP

    mesh = plsc.VectorSubcoreMesh(core_axis_name="c", subcore_axis_name="s")

    @functools.partial(
        pl.kernel, mesh=mesh,
        compiler_params=pltpu.CompilerParams(
            needs_layout_passes=False, use_tc_tiling_on_sc=False),
        out_type=(
            jax.ShapeDtypeStruct((E * 16,), jnp.float32),
            jax.ShapeDtypeStruct((NW, CT), jnp.float32),
        ),
        scratch_types=[
            pltpu.VMEM((ept,), jnp.int32),
            pltpu.VMEM((ept,), jnp.int32),
            pltpu.VMEM((CHUNK, 16), jnp.float32),
            pltpu.VMEM((CHUNK, 16), jnp.float32),
            pltpu.VMEM((CHUNK, 16), jnp.float32),
            pltpu.VMEM((CHUNK, 16), jnp.float32),
            pltpu.VMEM((CHUNK * 16,), jnp.float32),
            pltpu.VMEM((CHUNK * 16,), jnp.float32),
            pltpu.VMEM((CHUNK * 16,), jnp.float32),
            pltpu.VMEM((CHUNK * 16,), jnp.float32),
            pltpu.VMEM((CT,), jnp.float32),
            pltpu.SemaphoreType.DMA,
            pltpu.SemaphoreType.DMA,
            pltpu.SemaphoreType.DMA,
            pltpu.SemaphoreType.DMA,
            pltpu.SemaphoreType.DMA,
            pltpu.SemaphoreType.DMA,
        ],
    )
    def k(tab_h, src_h, dst_h, brows_h, coef_h,
          sidx, didx, srows0, srows1, drows0, drows1, st, dt, brow0, brow1,
          coefT, sa0, sa1, sb0, sb1, sw0, sw1):
        wid = lax.axis_index("s") * NC + lax.axis_index("c")
        pltpu.sync_copy(src_h.at[pl.ds(wid * ept, ept)], sidx)
        pltpu.sync_copy(dst_h.at[pl.ds(wid * ept, ept)], didx)

        zero = jnp.zeros((L,), jnp.float32)

        def zb(i, carry):
            brow0[pl.ds(i * L, L)] = zero
            brow1[pl.ds(i * L, L)] = zero
            return carry

        lax.fori_loop(0, (CHUNK * 16) // L, zb, 0)

        def zc(i, carry):
            for j in range(16):
                coefT[pl.ds((i * 16 + j) * L, L)] = zero
            return carry

        lax.fori_loop(0, CT // (16 * L), zc, 0)

        iota = lax.iota(jnp.int32, L)

        def issue(c, sbuf, dbuf, ssem, dsem):
            pltpu.async_copy(
                tab_h.at[sidx.at[pl.ds(c * CHUNK, CHUNK)]], sbuf, ssem)
            pltpu.async_copy(
                tab_h.at[didx.at[pl.ds(c * CHUNK, CHUNK)]], dbuf, dsem)

        def wait_rows(c, sbuf, dbuf, ssem, dsem):
            pltpu.make_async_copy(
                tab_h.at[sidx.at[pl.ds(c * CHUNK, CHUNK)]], sbuf, ssem).wait()
            pltpu.make_async_copy(
                tab_h.at[didx.at[pl.ds(c * CHUNK, CHUNK)]], dbuf, dsem).wait()

        def brow_dst(c):
            return brows_h.at[pl.ds((wid * ept + c * CHUNK) * 16, CHUNK * 16)]

        def compute(c, sbuf, dbuf, browbuf, wsem):
            def tr(eb, carry):
                for j in range(8):
                    e = eb * 8 + j
                    plsc.store_scatter(st, [iota * CHUNK + e], sbuf[e, :])
                    plsc.store_scatter(dt, [iota * CHUNK + e], dbuf[e, :])
                return carry

            lax.fori_loop(0, CHUNK // 8, tr, 0)
            for g in range(G):
                ridx = iota + g * L
                sa = [st[pl.ds(a * CHUNK + g * L, L)] for a in range(A)]
                sb = [dt[pl.ds((a + 5) * CHUNK + g * L, L)] for a in range(A)]
                lg = [v + u for v, u in zip(sa, sb)]
                lg = [jnp.maximum(v, v * 0.01) for v in lg]
                m = jnp.maximum(jnp.maximum(jnp.maximum(lg[0], lg[1]),
                                            jnp.maximum(lg[2], lg[3])), lg[4])
                ex = [jnp.exp(v - m) for v in lg]
                tot = ((ex[0] + ex[1]) + (ex[2] + ex[3])) + ex[4]
                inv = jnp.float32(1.0) / tot
                bb = [e * inv for e in ex]
                si = sidx[pl.ds(c * CHUNK + g * L, L)]
                di = didx[pl.ds(c * CHUNK + g * L, L)]
                for a in range(A):
                    plsc.store_scatter(browbuf, [ridx * 16 + a], bb[a])
                    plsc.addupdate_scatter(coefT, [si + a * NP], bb[a])
                    plsc.addupdate_scatter(coefT, [di + a * NP], bb[a])
            pltpu.async_copy(browbuf, brow_dst(c), wsem)

        def wait_brow(c_prev, browbuf, wsem):
            pltpu.make_async_copy(browbuf, brow_dst(c_prev), wsem).wait()

        issue(0, srows0, drows0, sa0, sb0)

        def pair_body(kk, carry):
            c0 = 2 * kk
            issue(c0 + 1, srows1, drows1, sa1, sb1)
            wait_rows(c0, srows0, drows0, sa0, sb0)

            @pl.when(kk > 0)
            def _():
                wait_brow(c0 - 2, brow0, sw0)

            compute(c0, srows0, drows0, brow0, sw0)
            issue(c0 + 2, srows0, drows0, sa0, sb0)
            wait_rows(c0 + 1, srows1, drows1, sa1, sb1)

            @pl.when(kk > 0)
            def _():
                wait_brow(c0 - 1, brow1, sw1)

            compute(c0 + 1, srows1, drows1, brow1, sw1)
            return carry

        lax.fori_loop(0, (nch - 1) // 2, pair_body, 0)
        last = nch - 1
        wait_rows(last, srows0, drows0, sa0, sb0)
        wait_brow(last - 2, brow0, sw0)
        compute(last, srows0, drows0, brow0, sw0)
        wait_brow(last - 1, brow1, sw1)
        wait_brow(last, brow0, sw0)
        pltpu.sync_copy(coefT, coef_h.at[wid])

    return k(tab, src1d, dst1d)


def kernel(x, edge_index, layer, W_att, b_att, node_anchor, W_ew, b_ew,
           edge_anchor, W_c1, b_c1, W_c2, b_c2, W_i, b_i):
    f32 = jnp.float32
    wcat = jnp.concatenate(
        [W_ew[:D], W_ew[D:], jnp.zeros((D, 6), f32)], axis=1)
    bcat = jnp.concatenate([b_ew, jnp.zeros((11,), f32)])[None, :]
    wap = jnp.concatenate([W_att, jnp.zeros((D, 3), f32)], axis=1)
    bap = jnp.concatenate([b_att, jnp.full((3,), NEG, f32)])[None, :]
    nap = jnp.concatenate([node_anchor, jnp.zeros((3, D), f32)], axis=0)
    eap8 = jnp.concatenate([edge_anchor, jnp.zeros((3, D), f32)], axis=0)
    eap16 = jnp.concatenate([edge_anchor, jnp.zeros((11, D), f32)], axis=0)
    w1a, w1b = W_c1[:D], W_c1[D:]
    bc1 = b_c1[None, :]
    bc2 = b_c2[None, :]
    bi = b_i[None, :]
    src1d = edge_index[0]
    dst1d = edge_index[1]

    tab = pl.pallas_call(
        _tab_body,
        grid=(5,),
        in_specs=[
            pl.BlockSpec((2000, D), lambda i: (i, 0)),
            pl.BlockSpec((D, 16), lambda i: (0, 0)),
            pl.BlockSpec((1, 16), lambda i: (0, 0)),
        ],
        out_specs=pl.BlockSpec((2000, 16), lambda i: (i, 0)),
        out_shape=jax.ShapeDtypeStruct((N, 16), f32),
    )(x, wcat, bcat)

    brows_flat, coef_flat = _edge_sc(tab, src1d, dst1d)
    brows8 = brows_flat.reshape(E // 8, 8 * 16)
    coef_part = coef_flat.reshape(-1, 8, NP)
    wbd = jnp.kron(jnp.eye(8, dtype=f32), eap16)

    edge_prompt = pl.pallas_call(
        _edge_prompt_body,
        grid=(100,),
        in_specs=[
            pl.BlockSpec((400, D), lambda i: (i, 0)),
            pl.BlockSpec((D, 8 * D), lambda i: (0, 0)),
        ],
        out_specs=pl.BlockSpec((3200, D), lambda i: (i, 0)),
        out_shape=jax.ShapeDtypeStruct((E, D), f32),
    )(brows8, wbd)

    NWP = coef_part.shape[0]
    nblk = (N + D - 1) // D
    final_x = pl.pallas_call(
        _final_body,
        grid=(nblk,),
        in_specs=[
            pl.BlockSpec((D, D), lambda i: (i, 0)),
            pl.BlockSpec((NWP, 8, D), lambda i: (0, 0, i)),
            pl.BlockSpec((D, 8), lambda i: (0, 0)),
            pl.BlockSpec((1, 8), lambda i: (0, 0)),
            pl.BlockSpec((8, D), lambda i: (0, 0)),
            pl.BlockSpec((8, D), lambda i: (0, 0)),
            pl.BlockSpec((D, D), lambda i: (0, 0)),
            pl.BlockSpec((D, D), lambda i: (0, 0)),
            pl.BlockSpec((1, D), lambda i: (0, 0)),
            pl.BlockSpec((D, D), lambda i: (0, 0)),
            pl.BlockSpec((1, D), lambda i: (0, 0)),
            pl.BlockSpec((D, D), lambda i: (0, 0)),
            pl.BlockSpec((1, D), lambda i: (0, 0)),
        ],
        out_specs=pl.BlockSpec((D, D), lambda i: (i, 0)),
        out_shape=jax.ShapeDtypeStruct((N, D), f32),
    )(x, coef_part, wap, bap, nap, eap8, w1a, w1b, bc1, W_c2, bc2, W_i, bi)

    return (final_x, edge_prompt)
